# Initial kernel scaffold; baseline (speedup 1.0000x reference)
import jax
import jax.numpy as jnp
from jax.experimental import pallas as pl


def kernel(particles, action, domain_id, params):
    raise NotImplementedError("write your pallas kernel here")


# speedup vs baseline: 1.8967x; 1.8967x over previous
import functools

import jax
import jax.numpy as jnp
from jax.experimental import pallas as pl
from jax.experimental.pallas import tpu as pltpu

D_MODEL = 768
D_FF = 3072
N_EXP = 8
TOPK = 2
N_HEADS = 12
EPS = 1e-5
F_BLK = 768


def _ln(x, g, b):
    m = x.mean(-1, keepdims=True)
    v = ((x - m) ** 2).mean(-1, keepdims=True)
    return (x - m) / jnp.sqrt(v + EPS) * g + b


def _mha(x, lp):
    Bq, T, D = x.shape
    H = N_HEADS
    hd = D // H
    q = (x @ lp['wq'] + lp['bq']).reshape(Bq, T, H, hd).transpose(0, 2, 1, 3)
    k = (x @ lp['wk'] + lp['bk']).reshape(Bq, T, H, hd).transpose(0, 2, 1, 3)
    v = (x @ lp['wv'] + lp['bv']).reshape(Bq, T, H, hd).transpose(0, 2, 1, 3)
    s = jnp.einsum('bhtd,bhsd->bhts', q, k) / jnp.sqrt(jnp.float32(hd))
    a = jax.nn.softmax(s, axis=-1)
    o = jnp.einsum('bhts,bhsd->bhtd', a, v).transpose(0, 2, 1, 3).reshape(Bq, T, D)
    return o @ lp['wo'] + lp['bo']


def _moe_body(gates_ref, x_ref, w1_ref, b1_ref, w2_ref, b2_ref, out_ref):
    e = pl.program_id(0)
    fb = pl.program_id(1)

    @pl.when((e == 0) & (fb == 0))
    def _init():
        out_ref[...] = jnp.zeros_like(out_ref)

    x = x_ref[...]
    h = jnp.dot(x, w1_ref[0], preferred_element_type=jnp.float32)
    h = h + b1_ref[0, 0]
    h = 0.5 * h * (1.0 + jax.lax.erf(h * 0.7071067811865476))
    o = jnp.dot(h, w2_ref[0], preferred_element_type=jnp.float32)
    g = gates_ref[0]
    out_ref[...] += g * o

    @pl.when(fb == 0)
    def _bias2():
        out_ref[...] += g * b2_ref[0, 0]


def _moe(x, lp):
    T = x.shape[0]
    logits = x @ lp['router']
    probs = jax.nn.softmax(logits, axis=-1)
    topk_probs, topk_idx = jax.lax.top_k(probs, TOPK)
    topk_probs = topk_probs / topk_probs.sum(-1, keepdims=True)
    gates = (jax.nn.one_hot(topk_idx, N_EXP, dtype=x.dtype)
             * topk_probs[..., None]).sum(1)
    gates_t = gates.T.reshape(N_EXP, T, 1)

    nfb = D_FF // F_BLK
    out = pl.pallas_call(
        _moe_body,
        grid=(N_EXP, nfb),
        in_specs=[
            pl.BlockSpec((1, T, 1), lambda e, fb: (e, 0, 0)),
            pl.BlockSpec((T, D_MODEL), lambda e, fb: (0, 0)),
            pl.BlockSpec((1, D_MODEL, F_BLK), lambda e, fb: (e, 0, fb)),
            pl.BlockSpec((1, 1, F_BLK), lambda e, fb: (e, 0, fb)),
            pl.BlockSpec((1, F_BLK, D_MODEL), lambda e, fb: (e, fb, 0)),
            pl.BlockSpec((1, 1, D_MODEL), lambda e, fb: (e, 0, 0)),
        ],
        out_specs=pl.BlockSpec((T, D_MODEL), lambda e, fb: (0, 0)),
        out_shape=jax.ShapeDtypeStruct((T, D_MODEL), x.dtype),
    )(gates_t, x, lp['w1'], lp['b1'].reshape(N_EXP, 1, D_FF),
      lp['w2'], lp['b2'].reshape(N_EXP, 1, D_MODEL))
    return out


def _forward(particles, action, domain_id, params):
    a = _ln(action @ params['ap_w'] + params['ap_b'], params['ap_g'], params['ap_be'])
    x = particles + a[:, None, :]
    x = x + params['dom'][domain_id][:, None, :]
    for lp in params['layers']:
        xn = _ln(x, lp['g1'], lp['b1n'])
        x = x + _mha(xn, lp)
        xn = _ln(x, lp['g2'], lp['b2n'])
        Bq, T, D = x.shape
        x = x + _moe(xn.reshape(Bq * T, D), lp).reshape(Bq, T, D)
    out = _ln(x, params['out_g'], params['out_bn'])
    return out @ params['op_w'] + params['op_b']


def kernel(particles, action, domain_id, params):
    return _forward(particles, action, domain_id, params)
